# baseline (speedup 1.0000x reference)
import functools

import numpy as np
import jax
import jax.numpy as jnp
from jax import lax
from jax.experimental import pallas as pl
from jax.experimental.pallas import tpu as pltpu
from jax.experimental.pallas import tpu_sc as plsc

L = 32
A = 32
B = 4096
N = A + L
C = N * (N - 1) // 2
OBS = 4 + 2 * L + 2 * (A - 1) + (A - 1)
F = 8

BB = 256
G = B // BB

_NW = 32
_BPW = B // _NW
_CB = 16
_CHUNKS = _BPW // _CB
_NT = B * N * F // 1024


def _edge_base():
    r, c = np.triu_indices(N, 1)
    rc = np.stack([r, c]).astype(np.int32)
    j = np.arange(BB * C)
    base = rc[:, j % C] + (N * (j // C)).astype(np.int32)[None, :]
    return np.ascontiguousarray(base, dtype=np.int32)


def _sc_tables():
    def rel_col(k, axis):
        if k == 0:
            return 0
        if k <= L:
            return 4 + 2 * (k - 1) + axis
        return 4 + 2 * L + 2 * (k - 1 - L) + axis

    idx = np.zeros((7, 4, 16), np.int32)
    for c in range(4):
        for lane in range(16):
            k = 16 * c + lane
            idx[0, c, lane] = rel_col(k, 0)
            idx[1, c, lane] = rel_col(k, 1)
            idx[2, c, lane] = rel_col(k, 0) if k else 0
            idx[3, c, lane] = rel_col(k, 1) if k else 1
            idx[4, c, lane] = rel_col(k, 0)
            idx[5, c, lane] = rel_col(k, 1)
            idx[6, c, lane] = 4 + 2 * L + 2 * (A - 1) + (k - 1 - L) if k > L else 0
    f32 = np.zeros((3, 16), np.float32)
    f32[0] = 1.0
    f32[0, 0] = 0.0
    f32[1, 0] = 2.0
    f32[2] = 1.0
    f32[2, 0] = 0.0
    return idx.reshape(-1), f32.reshape(-1)


_BASE_NP = _edge_base()
_IDX_NP, _F32_NP = _sc_tables()


def _tc_body(obs_ref, base_ref, ei_ref, ea_ref, bv_ref):
    i = pl.program_id(0)
    obs = obs_ref[...]
    velx, vely = obs[:, 0:1], obs[:, 1:2]
    posx, posy = obs[:, 2:3], obs[:, 3:4]
    rx = 1.0 / (0.001 + velx)
    ry = 1.0 / (0.001 + vely)

    def edge(cx):
        relx, rely = obs[:, cx:cx + 1], obs[:, cx + 1:cx + 2]
        ax, ay = posx + relx, posy + rely
        d0x, d0y = posx - ax, posy - ay
        d1x, d1y = velx - relx, vely - rely
        d2x, d2y = relx * rx, rely * ry
        s = (d0x * d0x + d0y * d0y + d1x * d1x + d1y * d1y
             + d2x * d2x + d2y * d2y + 4.0)
        return jnp.sqrt(s)

    ea_ref[...] = jnp.concatenate([edge(4), edge(6)], axis=1)
    bv_ref[...] = (jax.lax.iota(jnp.int32, BB * N) >> 6) + i * BB
    ei_ref[...] = base_ref[...] + i * (BB * N)


def _run_tc(obs, base):
    return pl.pallas_call(
        _tc_body,
        grid=(G,),
        in_specs=[
            pl.BlockSpec((BB, 8), lambda i: (i, 0)),
            pl.BlockSpec((2, BB * C), lambda i: (0, 0)),
        ],
        out_specs=[
            pl.BlockSpec((2, BB * C), lambda i: (0, i)),
            pl.BlockSpec((BB, 2), lambda i: (i, 0)),
            pl.BlockSpec((BB * N,), lambda i: (i,)),
        ],
        out_shape=[
            jax.ShapeDtypeStruct((2, B * C), jnp.int32),
            jax.ShapeDtypeStruct((B, 2), jnp.float32),
            jax.ShapeDtypeStruct((B * N,), jnp.int32),
        ],
    )(obs, base)


def _run_sc(obs, idx_tab, f32_tab):
    mesh = plsc.VectorSubcoreMesh(core_axis_name="c", subcore_axis_name="s")

    @functools.partial(
        pl.kernel,
        out_type=jax.ShapeDtypeStruct((B * N * F,), jnp.float32),
        mesh=mesh,
        compiler_params=pltpu.CompilerParams(
            needs_layout_passes=False, use_tc_tiling_on_sc=False),
        scratch_types=[
            pltpu.VMEM((_CB * OBS,), jnp.float32),
            pltpu.VMEM((_CB * OBS,), jnp.float32),
            pltpu.VMEM((7 * 4 * 16,), jnp.int32),
            pltpu.VMEM((3 * 16,), jnp.float32),
            pltpu.VMEM((8 * 8 * 128,), jnp.float32),
            pltpu.VMEM((8 * 8 * 128,), jnp.float32),
            pltpu.SemaphoreType.DMA,
            pltpu.SemaphoreType.DMA,
            pltpu.SemaphoreType.DMA,
            pltpu.SemaphoreType.DMA,
        ],
    )
    def k(obs_hbm, idx_hbm, f32_hbm, out_hbm,
          obs_v0, obs_v1, idx_v, f32_v, buf_v0, buf_v1, os0, os1, ws0, ws1):
        wid = lax.axis_index("s") * 2 + lax.axis_index("c")
        pltpu.sync_copy(idx_hbm, idx_v)
        pltpu.sync_copy(f32_hbm, f32_v)
        sel0 = f32_v[pl.ds(0, 16)]
        f6c0 = f32_v[pl.ds(16, 16)]
        f6c2 = f32_v[pl.ds(32, 16)]
        zeros16 = jnp.zeros((16,), jnp.float32)
        ones16 = jnp.full((16,), 1.0, jnp.float32)
        idx_tabs = [[idx_v[pl.ds(16 * (slot * 4 + c), 16)] for c in range(4)]
                    for slot in range(7)]
        obs_bufs = [obs_v0, obs_v1]
        out_bufs = [buf_v0, buf_v1]
        osems = [os0, os1]
        wsems = [ws0, ws1]

        def obs_fetch(ch, ob, sem):
            b0 = (wid * _BPW + ch * _CB) * OBS
            return pltpu.async_copy(obs_hbm.at[pl.ds(b0, _CB * OBS)], ob, sem)

        def compute_chunk(ob, buf_v):
            @plsc.parallel_loop(0, _CB, unroll=2)
            def batch_body(bl):
                roff = bl * OBS
                splat = jnp.full((16,), 0, jnp.int32) + roff
                velx = plsc.load_gather(ob, [splat])
                vely = plsc.load_gather(ob, [splat + 1])
                posx = plsc.load_gather(ob, [splat + 2])
                posy = plsc.load_gather(ob, [splat + 3])
                rx = 1.0 / (0.001 + velx)
                ry = 1.0 / (0.001 + vely)
                bbase = (bl // 2) * 1024 + (bl % 2) * 64
                for c in range(4):
                    off = bbase + 16 * c

                    def g(slot, c=c, roff=roff):
                        return plsc.load_gather(ob, [idx_tabs[slot][c] + roff])

                    a0, a1 = g(0), g(1)
                    if c == 0:
                        a0, a1 = a0 * sel0, a1 * sel0
                    buf_v[pl.ds(off, 16)] = a0 + posx
                    buf_v[pl.ds(off + 128, 16)] = a1 + posy
                    buf_v[pl.ds(off + 256, 16)] = g(2)
                    buf_v[pl.ds(off + 384, 16)] = g(3)
                    a4, a5 = g(4), g(5)
                    if c == 0:
                        a4, a5 = a4 * sel0, a5 * sel0
                    buf_v[pl.ds(off + 512, 16)] = a4 * rx
                    buf_v[pl.ds(off + 640, 16)] = a5 * ry
                    if c == 0:
                        buf_v[pl.ds(off + 768, 16)] = f6c0
                    elif c == 1:
                        buf_v[pl.ds(off + 768, 16)] = zeros16
                    elif c == 2:
                        buf_v[pl.ds(off + 768, 16)] = f6c2
                    else:
                        buf_v[pl.ds(off + 768, 16)] = ones16
                    if c < 2:
                        buf_v[pl.ds(off + 896, 16)] = zeros16
                    else:
                        a7 = g(6)
                        if c == 2:
                            a7 = a7 * sel0
                        buf_v[pl.ds(off + 896, 16)] = a7

        oh = [obs_fetch(0, obs_bufs[0], osems[0]), None]
        wh = [None, None]
        for ch in range(_CHUNKS):
            s = ch % 2
            if ch + 1 < _CHUNKS:
                oh[1 - s] = obs_fetch(ch + 1, obs_bufs[1 - s], osems[1 - s])
            oh[s].wait()
            if wh[s] is not None:
                wh[s].wait()
            compute_chunk(obs_bufs[s], out_bufs[s])
            wh[s] = pltpu.async_copy(
                out_bufs[s],
                out_hbm.at[pl.ds((wid * 64 + ch * 8) * 1024, 8 * 1024)],
                wsems[s])
        wh[0].wait()
        wh[1].wait()

    return k(obs, idx_tab, f32_tab)


def kernel(batch_observations):
    base = jnp.asarray(_BASE_NP)
    idx_tab = jnp.asarray(_IDX_NP)
    f32_tab = jnp.asarray(_F32_NP)
    ei, ea, bv = _run_tc(batch_observations[:, :8], base)
    xt = _run_sc(batch_observations.reshape(-1), idx_tab, f32_tab)
    x = jnp.transpose(xt.reshape(_NT, 8, 128), (0, 2, 1)).reshape(B * N, F)
    return x, ei, ea.reshape(-1), bv


# speedup vs baseline: 1.0390x; 1.0390x over previous
import functools

import numpy as np
import jax
import jax.numpy as jnp
from jax import lax
from jax.experimental import pallas as pl
from jax.experimental.pallas import tpu as pltpu
from jax.experimental.pallas import tpu_sc as plsc

L = 32
A = 32
B = 4096
N = A + L
C = N * (N - 1) // 2
OBS = 4 + 2 * L + 2 * (A - 1) + (A - 1)
F = 8

BB = 256
G = B // BB

_NW = 32
_BPW = B // _NW
_CB = 16
_CHUNKS = _BPW // _CB
_NT = B * N * F // 1024


def _edge_base():
    r, c = np.triu_indices(N, 1)
    rc = np.stack([r, c]).astype(np.int32)
    j = np.arange(BB * C)
    base = rc[:, j % C] + (N * (j // C)).astype(np.int32)[None, :]
    return np.ascontiguousarray(base, dtype=np.int32)


def _sc_tables():
    def rel_col(k, axis):
        if k == 0:
            return 0
        if k <= L:
            return 4 + 2 * (k - 1) + axis
        return 4 + 2 * L + 2 * (k - 1 - L) + axis

    idx = np.zeros((7, 4, 16), np.int32)
    for c in range(4):
        for lane in range(16):
            k = 16 * c + lane
            idx[0, c, lane] = rel_col(k, 0)
            idx[1, c, lane] = rel_col(k, 1)
            idx[2, c, lane] = rel_col(k, 0) if k else 0
            idx[3, c, lane] = rel_col(k, 1) if k else 1
            idx[4, c, lane] = rel_col(k, 0)
            idx[5, c, lane] = rel_col(k, 1)
            idx[6, c, lane] = 4 + 2 * L + 2 * (A - 1) + (k - 1 - L) if k > L else 0
    f32 = np.zeros((3, 16), np.float32)
    f32[0] = 1.0
    f32[0, 0] = 0.0
    f32[1, 0] = 2.0
    f32[2] = 1.0
    f32[2, 0] = 0.0
    return idx.reshape(-1), f32.reshape(-1)


_BASE_NP = _edge_base()
_IDX_NP, _F32_NP = _sc_tables()


def _tc_body(obs_ref, base_ref, ei_ref, ea_ref, bv_ref):
    i = pl.program_id(0)
    obs = obs_ref[...]
    velx, vely = obs[:, 0:1], obs[:, 1:2]
    posx, posy = obs[:, 2:3], obs[:, 3:4]
    rx = 1.0 / (0.001 + velx)
    ry = 1.0 / (0.001 + vely)

    def edge(cx):
        relx, rely = obs[:, cx:cx + 1], obs[:, cx + 1:cx + 2]
        ax, ay = posx + relx, posy + rely
        d0x, d0y = posx - ax, posy - ay
        d1x, d1y = velx - relx, vely - rely
        d2x, d2y = relx * rx, rely * ry
        s = (d0x * d0x + d0y * d0y + d1x * d1x + d1y * d1y
             + d2x * d2x + d2y * d2y + 4.0)
        return jnp.sqrt(s)

    ea_ref[...] = jnp.concatenate([edge(4), edge(6)], axis=1)
    bv_ref[...] = (jax.lax.iota(jnp.int32, BB * N) >> 6) + i * BB
    ei_ref[...] = base_ref[...] + i * (BB * N)


def _run_tc(obs, base):
    return pl.pallas_call(
        _tc_body,
        grid=(G,),
        in_specs=[
            pl.BlockSpec((BB, 8), lambda i: (i, 0)),
            pl.BlockSpec((2, BB * C), lambda i: (0, 0)),
        ],
        out_specs=[
            pl.BlockSpec((2, BB * C), lambda i: (0, i)),
            pl.BlockSpec((BB, 2), lambda i: (i, 0)),
            pl.BlockSpec((BB * N,), lambda i: (i,)),
        ],
        out_shape=[
            jax.ShapeDtypeStruct((2, B * C), jnp.int32),
            jax.ShapeDtypeStruct((B, 2), jnp.float32),
            jax.ShapeDtypeStruct((B * N,), jnp.int32),
        ],
    )(obs, base)


def _run_sc(obs, idx_tab, f32_tab):
    mesh = plsc.VectorSubcoreMesh(core_axis_name="c", subcore_axis_name="s")

    @functools.partial(
        pl.kernel,
        out_type=jax.ShapeDtypeStruct((B * N * F,), jnp.float32),
        mesh=mesh,
        compiler_params=pltpu.CompilerParams(
            needs_layout_passes=False, use_tc_tiling_on_sc=False),
        scratch_types=[
            pltpu.VMEM((_CB * OBS,), jnp.float32),
            pltpu.VMEM((_CB * OBS,), jnp.float32),
            pltpu.VMEM((7 * 4 * 16,), jnp.int32),
            pltpu.VMEM((3 * 16,), jnp.float32),
            pltpu.VMEM((8 * 8 * 128,), jnp.float32),
            pltpu.VMEM((8 * 8 * 128,), jnp.float32),
            pltpu.SemaphoreType.DMA,
            pltpu.SemaphoreType.DMA,
            pltpu.SemaphoreType.DMA,
            pltpu.SemaphoreType.DMA,
        ],
    )
    def k(obs_hbm, idx_hbm, f32_hbm, out_hbm,
          obs_v0, obs_v1, idx_v, f32_v, buf_v0, buf_v1, os0, os1, ws0, ws1):
        wid = lax.axis_index("s") * 2 + lax.axis_index("c")
        pltpu.sync_copy(idx_hbm, idx_v)
        pltpu.sync_copy(f32_hbm, f32_v)
        sel0 = f32_v[pl.ds(0, 16)]
        f6c0 = f32_v[pl.ds(16, 16)]
        f6c2 = f32_v[pl.ds(32, 16)]
        zeros16 = jnp.zeros((16,), jnp.float32)
        ones16 = jnp.full((16,), 1.0, jnp.float32)
        idx_tabs = [[idx_v[pl.ds(16 * (slot * 4 + c), 16)] for c in range(4)]
                    for slot in range(7)]
        obs_bufs = [obs_v0, obs_v1]
        out_bufs = [buf_v0, buf_v1]
        osems = [os0, os1]
        wsems = [ws0, ws1]

        def obs_fetch(ch, ob, sem):
            b0 = (wid * _BPW + ch * _CB) * OBS
            return pltpu.async_copy(obs_hbm.at[pl.ds(b0, _CB * OBS)], ob, sem)

        def compute_chunk(ob, buf_v):
            def batch_body(bl, carry2):
                roff = bl * OBS
                splat = jnp.full((16,), 0, jnp.int32) + roff
                velx = plsc.load_gather(ob, [splat])
                vely = plsc.load_gather(ob, [splat + 1])
                posx = plsc.load_gather(ob, [splat + 2])
                posy = plsc.load_gather(ob, [splat + 3])
                rx = 1.0 / (0.001 + velx)
                ry = 1.0 / (0.001 + vely)
                bbase = (bl // 2) * 1024 + (bl % 2) * 64
                for c in range(4):
                    off = bbase + 16 * c

                    def g(slot, c=c, roff=roff):
                        return plsc.load_gather(ob, [idx_tabs[slot][c] + roff])

                    a0, a1 = g(0), g(1)
                    if c == 0:
                        a0, a1 = a0 * sel0, a1 * sel0
                    buf_v[pl.ds(off, 16)] = a0 + posx
                    buf_v[pl.ds(off + 128, 16)] = a1 + posy
                    buf_v[pl.ds(off + 256, 16)] = g(2)
                    buf_v[pl.ds(off + 384, 16)] = g(3)
                    a4, a5 = g(4), g(5)
                    if c == 0:
                        a4, a5 = a4 * sel0, a5 * sel0
                    buf_v[pl.ds(off + 512, 16)] = a4 * rx
                    buf_v[pl.ds(off + 640, 16)] = a5 * ry
                    if c == 0:
                        buf_v[pl.ds(off + 768, 16)] = f6c0
                    elif c == 1:
                        buf_v[pl.ds(off + 768, 16)] = zeros16
                    elif c == 2:
                        buf_v[pl.ds(off + 768, 16)] = f6c2
                    else:
                        buf_v[pl.ds(off + 768, 16)] = ones16
                    if c < 2:
                        buf_v[pl.ds(off + 896, 16)] = zeros16
                    else:
                        a7 = g(6)
                        if c == 2:
                            a7 = a7 * sel0
                        buf_v[pl.ds(off + 896, 16)] = a7
                return carry2

            lax.fori_loop(0, _CB, batch_body, 0)

        oh = [obs_fetch(0, obs_bufs[0], osems[0]), None]
        wh = [None, None]
        for ch in range(_CHUNKS):
            s = ch % 2
            if ch + 1 < _CHUNKS:
                oh[1 - s] = obs_fetch(ch + 1, obs_bufs[1 - s], osems[1 - s])
            oh[s].wait()
            if wh[s] is not None:
                wh[s].wait()
            compute_chunk(obs_bufs[s], out_bufs[s])
            wh[s] = pltpu.async_copy(
                out_bufs[s],
                out_hbm.at[pl.ds((wid * 64 + ch * 8) * 1024, 8 * 1024)],
                wsems[s])
        wh[0].wait()
        wh[1].wait()

    return k(obs, idx_tab, f32_tab)


def kernel(batch_observations):
    base = jnp.asarray(_BASE_NP)
    idx_tab = jnp.asarray(_IDX_NP)
    f32_tab = jnp.asarray(_F32_NP)
    ei, ea, bv = _run_tc(batch_observations[:, :8], base)
    xt = _run_sc(batch_observations.reshape(-1), idx_tab, f32_tab)
    x = jnp.transpose(xt.reshape(_NT, 8, 128), (0, 2, 1)).reshape(B * N, F)
    return x, ei, ea.reshape(-1), bv
